# baseline (speedup 1.0000x reference)
import functools

import jax
import jax.numpy as jnp
from jax import lax
from jax.experimental import pallas as pl
from jax.experimental.pallas import tpu as pltpu

B = 128
N = 5000
E = 64
ROW_BLK = 8
CP_R = 1000


def _gather_body(uid_ref, iid_ref, um_blk, im_blk, um_out, im_out):
    b = pl.program_id(0)
    ru = lax.rem(uid_ref[b], ROW_BLK)
    ri = lax.rem(iid_ref[b], ROW_BLK)
    rows_iota = lax.broadcasted_iota(jnp.int32, (ROW_BLK, E), 0)
    um_row = jnp.sum(jnp.where(rows_iota == ru, um_blk[0], 0.0), axis=0,
                     keepdims=True)
    im_row = jnp.sum(jnp.where(rows_iota == ri, im_blk[0], 0.0), axis=0,
                     keepdims=True)
    um_out[0] = um_row
    im_out[0] = im_row


def _gather(user_memory, item_memory, uid, iid):
    grid_spec = pltpu.PrefetchScalarGridSpec(
        num_scalar_prefetch=2,
        grid=(B,),
        in_specs=[
            pl.BlockSpec((1, ROW_BLK, E),
                         lambda b, u, i: (b, u[b] // ROW_BLK, 0)),
            pl.BlockSpec((1, ROW_BLK, E),
                         lambda b, u, i: (b, i[b] // ROW_BLK, 0)),
        ],
        out_specs=[
            pl.BlockSpec((1, 1, E), lambda b, u, i: (b, 0, 0)),
            pl.BlockSpec((1, 1, E), lambda b, u, i: (b, 0, 0)),
        ],
    )
    um3, im3 = pl.pallas_call(
        _gather_body,
        grid_spec=grid_spec,
        out_shape=[
            jax.ShapeDtypeStruct((B, 1, E), jnp.float32),
            jax.ShapeDtypeStruct((B, 1, E), jnp.float32),
        ],
    )(uid, iid, user_memory, item_memory)
    return um3.reshape(B, E), im3.reshape(B, E)


def _gru_body(um_ref, im_ref, wih_u_ref, whh_u_ref, bih_u_ref, bhh_u_ref,
              wih_i_ref, whh_i_ref, bih_i_ref, bhh_i_ref,
              new_u_ref, new_i_ref):
    um = um_ref[...]
    im = im_ref[...]
    x_u = jnp.concatenate([um, im], axis=1)
    x_i = jnp.concatenate([im, um], axis=1)

    def cell(x, h, wih, whh, bih, bhh):
        gi = lax.dot_general(x, wih, (((1,), (1,)), ((), ())),
                             preferred_element_type=jnp.float32) + bih
        gh = lax.dot_general(h, whh, (((1,), (1,)), ((), ())),
                             preferred_element_type=jnp.float32) + bhh
        i_r, i_z, i_n = gi[:, :E], gi[:, E:2 * E], gi[:, 2 * E:]
        h_r, h_z, h_n = gh[:, :E], gh[:, E:2 * E], gh[:, 2 * E:]
        r = jax.nn.sigmoid(i_r + h_r)
        z = jax.nn.sigmoid(i_z + h_z)
        n = jnp.tanh(i_n + r * h_n)
        h2 = (1.0 - z) * n + z * h
        nrm = jnp.sqrt(jnp.sum(h2 * h2, axis=1, keepdims=True))
        return h2 / jnp.maximum(nrm, 1e-12)

    new_u_ref[...] = cell(x_u, um, wih_u_ref[...], whh_u_ref[...],
                          bih_u_ref[...], bhh_u_ref[...])
    new_i_ref[...] = cell(x_i, im, wih_i_ref[...], whh_i_ref[...],
                          bih_i_ref[...], bhh_i_ref[...])


def _gru(um, im, Wih_u, Whh_u, bih_u, bhh_u, Wih_i, Whh_i, bih_i, bhh_i):
    args = (um, im, Wih_u, Whh_u, bih_u, bhh_u, Wih_i, Whh_i, bih_i, bhh_i)
    return pl.pallas_call(
        _gru_body,
        in_specs=[pl.BlockSpec(x.shape, lambda *_: (0,) * x.ndim)
                  for x in args],
        out_specs=[pl.BlockSpec((B, E), lambda: (0, 0)),
                   pl.BlockSpec((B, E), lambda: (0, 0))],
        out_shape=[jax.ShapeDtypeStruct((B, E), jnp.float32),
                   jax.ShapeDtypeStruct((B, E), jnp.float32)],
    )(*args)


def _copy_body(ids_ref, mem_blk, new_ref, out_blk):
    b = pl.program_id(0)
    j = pl.program_id(1)
    target = ids_ref[b]
    r0 = j * CP_R
    rows = lax.broadcasted_iota(jnp.int32, (CP_R, 1), 0) + r0
    new_row = new_ref[pl.ds(b, 1), :]
    out_blk[0] = jnp.where(rows == target, new_row, mem_blk[0])


def _copy_scatter(memory, new_rows, ids):
    grid_spec = pltpu.PrefetchScalarGridSpec(
        num_scalar_prefetch=1,
        grid=(B, N // CP_R),
        in_specs=[
            pl.BlockSpec((1, CP_R, E), lambda b, j, ids: (b, j, 0)),
            pl.BlockSpec((B, E), lambda b, j, ids: (0, 0)),
        ],
        out_specs=pl.BlockSpec((1, CP_R, E), lambda b, j, ids: (b, j, 0)),
    )
    return pl.pallas_call(
        _copy_body,
        grid_spec=grid_spec,
        out_shape=jax.ShapeDtypeStruct((B, N, E), jnp.float32),
        compiler_params=pltpu.CompilerParams(
            dimension_semantics=("parallel", "parallel")),
    )(ids, memory, new_rows)


def kernel(user_ids, item_ids, user_features, item_features,
           user_memory, item_memory,
           Wih_u, Whh_u, bih_u, bhh_u, Wih_i, Whh_i, bih_i, bhh_i):
    uid = user_ids.astype(jnp.int32)
    iid = item_ids.astype(jnp.int32)

    um, im = _gather(user_memory, item_memory, uid, iid)
    new_u, new_i = _gru(um, im, Wih_u, Whh_u,
                        bih_u.reshape(1, 3 * E), bhh_u.reshape(1, 3 * E),
                        Wih_i, Whh_i,
                        bih_i.reshape(1, 3 * E), bhh_i.reshape(1, 3 * E))
    new_user_memory = _copy_scatter(user_memory, new_u, uid)
    new_item_memory = _copy_scatter(item_memory, new_i, iid)

    out = jnp.concatenate([
        user_ids[:, None].astype(jnp.float32),
        item_ids[:, None].astype(jnp.float32),
        new_u,
        new_i,
    ], axis=1)
    return out, new_user_memory, new_item_memory
